# baseline (speedup 1.0000x reference)
import jax
import jax.numpy as jnp
from jax import lax
from jax.experimental import pallas as pl
from jax.experimental.pallas import tpu as pltpu

N = 4096
D1, D2, D3, D_IN = 128, 256, 512, 512
PB = 512
TM = 1024


def _body(z_ref, adj_ref, w4_ref, w5_ref, w6_ref,
          zhat_ref, recon_ref,
          abuf_ref, s1_ref, s2_ref, s3_ref, zh_ref, zstage_ref, rbuf_ref,
          in_sem, zh_sem, out_sem):
    np_ = N // PB
    half = np_ // 2
    total = 3 * np_
    tj = N // TM
    bf = jnp.bfloat16

    hp = PB // 2

    def in_copy_h(g, slot, h):
        k = lax.rem(g, np_)
        return pltpu.make_async_copy(
            adj_ref.at[pl.ds(k * PB + h * hp, hp), :],
            abuf_ref.at[slot, pl.ds(h * hp, hp), :],
            in_sem.at[slot, h],
        )

    def in_start(g, slot):
        in_copy_h(g, slot, 0).start()
        in_copy_h(g, slot, 1).start()

    def in_wait(g, slot):
        in_copy_h(g, slot, 0).wait()
        in_copy_h(g, slot, 1).wait()

    in_start(0, 0)
    in_start(1, 1)

    acc = jnp.dot(z_ref[...].astype(bf), w4_ref[...].astype(bf),
                  preferred_element_type=jnp.float32)
    s1_ref[...] = jnp.tanh(acc).astype(bf)

    w5 = w5_ref[...].astype(bf)
    w6 = w6_ref[...].astype(bf)

    def compute_panel(p, k, a, kslot):
        rows = pl.ds(k * PB, PB)
        if p == 0:
            acc = jnp.dot(a, s1_ref[...], preferred_element_type=jnp.float32)
            r = jnp.dot(acc.astype(bf), w5, preferred_element_type=jnp.float32)
            s2_ref[rows, :] = jnp.tanh(r).astype(bf)
        elif p == 1:
            acc = jnp.dot(a, s2_ref[...], preferred_element_type=jnp.float32)
            r = jnp.dot(acc.astype(bf), w6, preferred_element_type=jnp.float32)
            s3_ref[rows, :] = r.astype(bf)
        else:
            acc = jnp.dot(a, s3_ref[...], preferred_element_type=jnp.float32)

            @pl.when(k >= 2)
            def _():
                zh_copy(k - 2, kslot).wait()

            zstage_ref[kslot] = acc
            zh_copy(k, kslot).start()
            zh_ref[rows, :] = acc.astype(bf)

    def zh_copy(m, slot):
        return pltpu.make_async_copy(
            zstage_ref.at[slot],
            zhat_ref.at[pl.ds(m * PB, PB), :],
            zh_sem.at[slot],
        )

    def make_pass(p):
        def body_fn(i, carry):
            g0 = p * np_ + 2 * i
            k0 = 2 * i

            in_wait(g0, 0)
            compute_panel(p, k0, abuf_ref[0].astype(bf), 0)

            @pl.when(g0 + 2 < total)
            def _():
                in_start(g0 + 2, 0)

            in_wait(g0 + 1, 1)
            compute_panel(p, k0 + 1, abuf_ref[1].astype(bf), 1)

            @pl.when(g0 + 3 < total)
            def _():
                in_start(g0 + 3, 1)

            return carry
        return body_fn

    for p in range(3):
        lax.fori_loop(0, half, make_pass(p), 0, unroll=False)

    zh_copy(np_ - 2, 0).wait()
    zh_copy(np_ - 1, 1).wait()

    hm = TM // 2

    def out_copy_h(t, slot, h):
        i = t // tj
        j = lax.rem(t, tj)
        return pltpu.make_async_copy(
            rbuf_ref.at[slot, pl.ds(h * hm, hm), :],
            recon_ref.at[pl.ds(i * TM + h * hm, hm), pl.ds(j * TM, TM)],
            out_sem.at[slot, h],
        )

    class _OutPair:
        def __init__(self, t, slot):
            self.t, self.slot = t, slot

        def start(self):
            out_copy_h(self.t, self.slot, 0).start()
            out_copy_h(self.t, self.slot, 1).start()

        def wait(self):
            out_copy_h(self.t, self.slot, 0).wait()
            out_copy_h(self.t, self.slot, 1).wait()

    def out_copy(t, slot):
        return _OutPair(t, slot)

    n_tiles = tj * tj

    def recon_tile(t, slot):
        i = t // tj
        j = lax.rem(t, tj)
        a = zh_ref[pl.ds(i * TM, TM), :]
        b = zh_ref[pl.ds(j * TM, TM), :]
        acc = lax.dot_general(
            a, b, dimension_numbers=(((1,), (1,)), ((), ())),
            preferred_element_type=jnp.float32)
        rbuf_ref[slot] = 0.5 * (1.0 + jnp.tanh(0.5 * acc))
        out_copy(t, slot).start()

    def recon_body(u, carry):
        t0 = 2 * u

        @pl.when(u >= 1)
        def _():
            out_copy(t0 - 2, 0).wait()

        recon_tile(t0, 0)

        @pl.when(u >= 1)
        def _():
            out_copy(t0 - 1, 1).wait()

        recon_tile(t0 + 1, 1)
        return carry

    lax.fori_loop(0, n_tiles // 2, recon_body, 0, unroll=False)
    out_copy(n_tiles - 2, 0).wait()
    out_copy(n_tiles - 1, 1).wait()


def kernel(z_igae, adj, W4, W5, W6):
    z_hat, z_hat_adj = pl.pallas_call(
        _body,
        in_specs=[
            pl.BlockSpec(memory_space=pltpu.VMEM),
            pl.BlockSpec(memory_space=pl.ANY),
            pl.BlockSpec(memory_space=pltpu.VMEM),
            pl.BlockSpec(memory_space=pltpu.VMEM),
            pl.BlockSpec(memory_space=pltpu.VMEM),
        ],
        out_specs=[
            pl.BlockSpec(memory_space=pl.ANY),
            pl.BlockSpec(memory_space=pl.ANY),
        ],
        out_shape=[
            jax.ShapeDtypeStruct((N, D_IN), jnp.float32),
            jax.ShapeDtypeStruct((N, N), jnp.float32),
        ],
        scratch_shapes=[
            pltpu.VMEM((2, PB, N), jnp.float32),
            pltpu.VMEM((N, D2), jnp.bfloat16),
            pltpu.VMEM((N, D3), jnp.bfloat16),
            pltpu.VMEM((N, D_IN), jnp.bfloat16),
            pltpu.VMEM((N, D_IN), jnp.bfloat16),
            pltpu.VMEM((2, PB, D_IN), jnp.float32),
            pltpu.VMEM((2, TM, TM), jnp.float32),
            pltpu.SemaphoreType.DMA((2, 2)),
            pltpu.SemaphoreType.DMA((2,)),
            pltpu.SemaphoreType.DMA((2, 2)),
        ],
    )(z_igae, adj, W4, W5, W6)
    return (z_hat, z_hat_adj)


# speedup vs baseline: 1.0013x; 1.0013x over previous
import jax
import jax.numpy as jnp
from jax import lax
from jax.experimental import pallas as pl
from jax.experimental.pallas import tpu as pltpu

N = 4096
D1, D2, D3, D_IN = 128, 256, 512, 512
PB = 512
TM = 1024


def _body(z_ref, adj_ref, w4_ref, w5_ref, w6_ref,
          zhat_ref, recon_ref,
          abuf_ref, s1_ref, s2_ref, s3_ref, zh_ref, zstage_ref, rbuf_ref,
          in_sem, zh_sem, out_sem):
    np_ = N // PB
    half = np_ // 2
    total = 3 * np_
    tj = N // TM
    bf = jnp.bfloat16

    def in_copy(g, slot):
        k = lax.rem(g, np_)
        return pltpu.make_async_copy(
            adj_ref.at[pl.ds(k * PB, PB), :],
            abuf_ref.at[slot],
            in_sem.at[slot],
        )

    in_copy(0, 0).start()
    in_copy(1, 1).start()

    acc = jnp.dot(z_ref[...].astype(bf), w4_ref[...].astype(bf),
                  preferred_element_type=jnp.float32)
    s1_ref[...] = jnp.tanh(acc).astype(bf)

    w5 = w5_ref[...].astype(bf)
    w6 = w6_ref[...].astype(bf)

    def compute_panel(p, k, a, kslot):
        rows = pl.ds(k * PB, PB)
        if p == 0:
            acc = jnp.dot(a, s1_ref[...], preferred_element_type=jnp.float32)
            r = jnp.dot(acc.astype(bf), w5, preferred_element_type=jnp.float32)
            s2_ref[rows, :] = jnp.tanh(r).astype(bf)
        elif p == 1:
            acc = jnp.dot(a, s2_ref[...], preferred_element_type=jnp.float32)
            r = jnp.dot(acc.astype(bf), w6, preferred_element_type=jnp.float32)
            s3_ref[rows, :] = r.astype(bf)
        else:
            acc = jnp.dot(a, s3_ref[...], preferred_element_type=jnp.float32)

            @pl.when(k >= 2)
            def _():
                zh_copy(k - 2, kslot).wait()

            zstage_ref[kslot] = acc
            zh_copy(k, kslot).start()
            zh_ref[rows, :] = acc.astype(bf)

    def zh_copy(m, slot):
        return pltpu.make_async_copy(
            zstage_ref.at[slot],
            zhat_ref.at[pl.ds(m * PB, PB), :],
            zh_sem.at[slot],
        )

    def make_pass(p):
        def body_fn(i, carry):
            g0 = p * np_ + 2 * i
            k0 = 2 * i

            in_copy(g0, 0).wait()
            compute_panel(p, k0, abuf_ref[0].astype(bf), 0)

            @pl.when(g0 + 2 < total)
            def _():
                in_copy(g0 + 2, 0).start()

            in_copy(g0 + 1, 1).wait()
            compute_panel(p, k0 + 1, abuf_ref[1].astype(bf), 1)

            @pl.when(g0 + 3 < total)
            def _():
                in_copy(g0 + 3, 1).start()

            return carry
        return body_fn

    for p in range(3):
        with jax.named_scope("adjpass%d" % p):
            lax.fori_loop(0, half, make_pass(p), 0, unroll=False)

    zh_copy(np_ - 2, 0).wait()
    zh_copy(np_ - 1, 1).wait()

    def out_copy(t, slot):
        i = t // tj
        j = lax.rem(t, tj)
        return pltpu.make_async_copy(
            rbuf_ref.at[slot],
            recon_ref.at[pl.ds(i * TM, TM), pl.ds(j * TM, TM)],
            out_sem.at[slot],
        )

    n_tiles = tj * tj

    def recon_tile(t, slot):
        i = t // tj
        j = lax.rem(t, tj)
        a = zh_ref[pl.ds(i * TM, TM), :]
        b = zh_ref[pl.ds(j * TM, TM), :]
        acc = lax.dot_general(
            a, b, dimension_numbers=(((1,), (1,)), ((), ())),
            preferred_element_type=jnp.float32)
        rbuf_ref[slot] = 0.5 * (1.0 + jnp.tanh(0.5 * acc))
        out_copy(t, slot).start()

    def recon_body(u, carry):
        t0 = 2 * u

        @pl.when(u >= 1)
        def _():
            out_copy(t0 - 2, 0).wait()

        recon_tile(t0, 0)

        @pl.when(u >= 1)
        def _():
            out_copy(t0 - 1, 1).wait()

        recon_tile(t0 + 1, 1)
        return carry

    with jax.named_scope("reconpass"):
        lax.fori_loop(0, n_tiles // 2, recon_body, 0, unroll=False)
    out_copy(n_tiles - 2, 0).wait()
    out_copy(n_tiles - 1, 1).wait()


def kernel(z_igae, adj, W4, W5, W6):
    z_hat, z_hat_adj = pl.pallas_call(
        _body,
        in_specs=[
            pl.BlockSpec(memory_space=pltpu.VMEM),
            pl.BlockSpec(memory_space=pl.ANY),
            pl.BlockSpec(memory_space=pltpu.VMEM),
            pl.BlockSpec(memory_space=pltpu.VMEM),
            pl.BlockSpec(memory_space=pltpu.VMEM),
        ],
        out_specs=[
            pl.BlockSpec(memory_space=pl.ANY),
            pl.BlockSpec(memory_space=pl.ANY),
        ],
        out_shape=[
            jax.ShapeDtypeStruct((N, D_IN), jnp.float32),
            jax.ShapeDtypeStruct((N, N), jnp.float32),
        ],
        scratch_shapes=[
            pltpu.VMEM((2, PB, N), jnp.float32),
            pltpu.VMEM((N, D2), jnp.bfloat16),
            pltpu.VMEM((N, D3), jnp.bfloat16),
            pltpu.VMEM((N, D_IN), jnp.bfloat16),
            pltpu.VMEM((N, D_IN), jnp.bfloat16),
            pltpu.VMEM((2, PB, D_IN), jnp.float32),
            pltpu.VMEM((2, TM, TM), jnp.float32),
            pltpu.SemaphoreType.DMA((2,)),
            pltpu.SemaphoreType.DMA((2,)),
            pltpu.SemaphoreType.DMA((2,)),
        ],
    )(z_igae, adj, W4, W5, W6)
    return (z_hat, z_hat_adj)
